# baseline (speedup 1.0000x reference)
import functools

import jax
import jax.numpy as jnp
from jax.experimental import pallas as pl
from jax.experimental.pallas import tpu as pltpu

_EPS = 1e-5
_VMEM = 48 * 1024 * 1024


def _cparams():
    return pltpu.CompilerParams(dimension_semantics=("parallel",),
                                vmem_limit_bytes=_VMEM)


def _taps(w_oihw):
    w = jnp.transpose(w_oihw, (2, 3, 1, 0)).astype(jnp.float32)
    return jnp.stack(
        [jnp.concatenate([w[dy, 0], w[dy, 1], w[dy, 2]], axis=0)
         for dy in range(3)], axis=0)


def _conv_stats_pool(cat_ref, w_ref, b_ref, mxo_ref, mno_ref, so_ref, qo_ref,
                     *, h, w, wa, cout):
    bb = cat_ref.shape[0]
    k = cat_ref.shape[-1]
    ho, wo = h - 2, w - 2
    hp, wp = ho // 2, wo // 2
    acc = None
    for dy in range(3):
        lhs = cat_ref[:, dy:dy + ho, :, :].reshape(bb * ho * wa, k)
        p = jnp.dot(lhs, w_ref[dy], preferred_element_type=jnp.float32)
        acc = p if acc is None else acc + p
    y = (acc + b_ref[...]).reshape(bb, ho, wa, cout)
    yv = y[:, :, 0:wo, :]
    so_ref[...] = jnp.sum(yv, axis=(0, 1, 2)).reshape(1, 1, cout)
    qo_ref[...] = jnp.sum(yv * yv, axis=(0, 1, 2)).reshape(1, 1, cout)
    t = yv.reshape(bb, hp, 2, wo, cout)
    rmax = jnp.max(t, axis=2).reshape(bb, hp, wp, 2, cout)
    rmin = jnp.min(t, axis=2).reshape(bb, hp, wp, 2, cout)
    mxo_ref[...] = jnp.max(rmax, axis=3)
    mno_ref[...] = jnp.min(rmin, axis=3)


def _l1_kernel(x_ref, w_ref, b_ref, mxo_ref, mno_ref, so_ref, qo_ref, cat_ref,
               *, h, w, wa, cin, cout):
    xv = x_ref[...]
    cat_ref[...] = jnp.zeros(cat_ref.shape, jnp.float32)
    cat_ref[:, :, :, 0:cin] = xv
    for j in (1, 2):
        cat_ref[:, :, 0:wa - j, j * cin:(j + 1) * cin] = xv[:, :, j:wa, :]
    _conv_stats_pool(cat_ref, w_ref, b_ref, mxo_ref, mno_ref, so_ref, qo_ref,
                     h=h, w=w, wa=wa, cout=cout)


def _bn_affine(s_ref, q_ref, g_ref, be_ref, n_prev, cin):
    inv = 1.0 / n_prev
    mean = jnp.sum(s_ref[...], axis=(0, 1)).reshape(1, cin) * inv
    ex2 = jnp.sum(q_ref[...], axis=(0, 1)).reshape(1, cin) * inv
    var = jnp.maximum(ex2 - mean * mean, 0.0)
    scale = g_ref[...] * jax.lax.rsqrt(var + _EPS)
    shift = be_ref[...] - mean * scale
    return scale.reshape(1, 1, 1, cin), shift.reshape(1, 1, 1, cin)


def _mid_kernel(mx_ref, mn_ref, s_ref, q_ref, g_ref, be_ref, w_ref, b_ref,
                mxo_ref, mno_ref, so_ref, qo_ref, cat_ref,
                *, n_prev, h, w, wa, cin, cout):
    sc, sh = _bn_affine(s_ref, q_ref, g_ref, be_ref, n_prev, cin)
    v = jnp.where(sc >= 0.0, mx_ref[...], mn_ref[...])
    act = jnp.maximum(v * sc + sh, 0.0)
    cat_ref[...] = jnp.zeros(cat_ref.shape, jnp.float32)
    cat_ref[:, :, 0:w, 0:cin] = act
    for j in (1, 2):
        cat_ref[:, :, 0:w - j, j * cin:(j + 1) * cin] = act[:, :, j:w, :]
    _conv_stats_pool(cat_ref, w_ref, b_ref, mxo_ref, mno_ref, so_ref, qo_ref,
                     h=h, w=w, wa=wa, cout=cout)


def _final_bn_kernel(mx_ref, mn_ref, s_ref, q_ref, g_ref, be_ref, o_ref,
                     *, n_prev, cin):
    sc, sh = _bn_affine(s_ref, q_ref, g_ref, be_ref, n_prev, cin)
    v = jnp.where(sc >= 0.0, mx_ref[...], mn_ref[...])
    o_ref[...] = jnp.maximum(v * sc + sh, 0.0)


def _mlp_kernel(x_ref, w1_ref, b1_ref, w2_ref, b2_ref, o_ref):
    hid = jnp.dot(x_ref[...], w1_ref[...],
                  preferred_element_type=jnp.float32) + b1_ref[...]
    hid = jnp.maximum(hid, 0.0)
    o_ref[...] = jnp.dot(hid, w2_ref[...],
                         preferred_element_type=jnp.float32) + b2_ref[...]


def _conv_layer(first, ins, wtap, bias, bn, *, n, h, w, wa, cin, cout, bb,
                n_prev):
    g = n // bb
    ho, wo = h - 2, w - 2
    hp, wp = ho // 2, wo // 2
    out_shape = (jax.ShapeDtypeStruct((n, hp, wp, cout), jnp.float32),
                 jax.ShapeDtypeStruct((n, hp, wp, cout), jnp.float32),
                 jax.ShapeDtypeStruct((g, 1, cout), jnp.float32),
                 jax.ShapeDtypeStruct((g, 1, cout), jnp.float32))
    out_specs = (pl.BlockSpec((bb, hp, wp, cout), lambda i: (i, 0, 0, 0)),
                 pl.BlockSpec((bb, hp, wp, cout), lambda i: (i, 0, 0, 0)),
                 pl.BlockSpec((1, 1, cout), lambda i: (i, 0, 0)),
                 pl.BlockSpec((1, 1, cout), lambda i: (i, 0, 0)))
    wspecs = [pl.BlockSpec((3, 3 * cin, cout), lambda i: (0, 0, 0)),
              pl.BlockSpec((1, cout), lambda i: (0, 0))]
    scratch = [pltpu.VMEM((bb, h, wa, 3 * cin), jnp.float32)]
    if first:
        x, = ins
        return pl.pallas_call(
            functools.partial(_l1_kernel, h=h, w=w, wa=wa, cin=cin, cout=cout),
            out_shape=out_shape,
            grid=(g,),
            in_specs=[pl.BlockSpec((bb, h, wa, cin), lambda i: (i, 0, 0, 0))]
            + wspecs,
            out_specs=out_specs,
            scratch_shapes=scratch,
            compiler_params=_cparams(),
        )(x, wtap, bias)
    mx, mn, s, q = ins
    gp = s.shape[0]
    gamma, beta = bn
    return pl.pallas_call(
        functools.partial(_mid_kernel, n_prev=n_prev, h=h, w=w, wa=wa,
                          cin=cin, cout=cout),
        out_shape=out_shape,
        grid=(g,),
        in_specs=[pl.BlockSpec((bb, h, w, cin), lambda i: (i, 0, 0, 0)),
                  pl.BlockSpec((bb, h, w, cin), lambda i: (i, 0, 0, 0)),
                  pl.BlockSpec((gp, 1, cin), lambda i: (0, 0, 0)),
                  pl.BlockSpec((gp, 1, cin), lambda i: (0, 0, 0)),
                  pl.BlockSpec((1, cin), lambda i: (0, 0)),
                  pl.BlockSpec((1, cin), lambda i: (0, 0))] + wspecs,
        out_specs=out_specs,
        scratch_shapes=scratch,
        compiler_params=_cparams(),
    )(mx, mn, s, q, gamma.astype(jnp.float32).reshape(1, cin),
      beta.astype(jnp.float32).reshape(1, cin), wtap, bias)


def kernel(x, w1, b1, w2, b2, w3, b3, w4, b4, g32, be32, g128, be128,
           g10, be10, lw1, lb1, lw2, lb2):
    n = x.shape[0]
    xh = jnp.transpose(x, (0, 2, 3, 1)).astype(jnp.float32)
    xh = jnp.pad(xh, ((0, 0), (0, 0), (0, 2), (0, 0)))

    o1 = _conv_layer(True, (xh,), _taps(w1),
                     b1.astype(jnp.float32).reshape(1, 32), None,
                     n=n, h=94, w=94, wa=96, cin=3, cout=32, bb=2, n_prev=0.0)
    o2 = _conv_layer(False, o1, _taps(w2),
                     b2.astype(jnp.float32).reshape(1, 128), (g32, be32),
                     n=n, h=46, w=46, wa=48, cin=32, cout=128, bb=4,
                     n_prev=float(n * 92 * 92))
    o3 = _conv_layer(False, o2, _taps(w3),
                     b3.astype(jnp.float32).reshape(1, 32), (g128, be128),
                     n=n, h=22, w=22, wa=24, cin=128, cout=32, bb=8,
                     n_prev=float(n * 44 * 44))
    o4 = _conv_layer(False, o3, _taps(w4),
                     b4.astype(jnp.float32).reshape(1, 10), (g32, be32),
                     n=n, h=10, w=10, wa=16, cin=32, cout=10, bb=32,
                     n_prev=float(n * 20 * 20))

    mx4, mn4, s4, q4 = o4
    g4 = s4.shape[0]
    act4 = pl.pallas_call(
        functools.partial(_final_bn_kernel, n_prev=float(n * 8 * 8), cin=10),
        out_shape=jax.ShapeDtypeStruct((n, 4, 4, 10), jnp.float32),
        grid=(1,),
        in_specs=[pl.BlockSpec((n, 4, 4, 10), lambda i: (0, 0, 0, 0)),
                  pl.BlockSpec((n, 4, 4, 10), lambda i: (0, 0, 0, 0)),
                  pl.BlockSpec((g4, 1, 10), lambda i: (0, 0, 0)),
                  pl.BlockSpec((g4, 1, 10), lambda i: (0, 0, 0)),
                  pl.BlockSpec((1, 10), lambda i: (0, 0)),
                  pl.BlockSpec((1, 10), lambda i: (0, 0))],
        out_specs=pl.BlockSpec((n, 4, 4, 10), lambda i: (0, 0, 0, 0)),
        compiler_params=_cparams(),
    )(mx4, mn4, s4, q4, g10.astype(jnp.float32).reshape(1, 10),
      be10.astype(jnp.float32).reshape(1, 10))

    rows = (n * 160) // 6400
    flat = jnp.transpose(act4, (0, 3, 1, 2)).reshape(rows, 6400)
    return pl.pallas_call(
        _mlp_kernel,
        out_shape=jax.ShapeDtypeStruct((rows, 6), jnp.float32),
        grid=(1,),
        in_specs=[pl.BlockSpec((rows, 6400), lambda i: (0, 0)),
                  pl.BlockSpec((6400, 100), lambda i: (0, 0)),
                  pl.BlockSpec((1, 100), lambda i: (0, 0)),
                  pl.BlockSpec((100, 6), lambda i: (0, 0)),
                  pl.BlockSpec((1, 6), lambda i: (0, 0))],
        out_specs=pl.BlockSpec((rows, 6), lambda i: (0, 0)),
        compiler_params=pltpu.CompilerParams(vmem_limit_bytes=_VMEM),
    )(flat, jnp.transpose(lw1).astype(jnp.float32),
      lb1.astype(jnp.float32).reshape(1, 100),
      jnp.transpose(lw2).astype(jnp.float32),
      lb2.astype(jnp.float32).reshape(1, 6))
